# Initial kernel scaffold; baseline (speedup 1.0000x reference)
import jax
import jax.numpy as jnp
from jax.experimental import pallas as pl


def kernel(mcc_code, tr_type, emb_mcc, emb_tr):
    raise NotImplementedError("write your pallas kernel here")


# speedup vs baseline: 6.3667x; 6.3667x over previous
import functools

import jax
import jax.numpy as jnp
from jax import lax
from jax.experimental import pallas as pl
from jax.experimental.pallas import tpu as pltpu
from jax.experimental.pallas import tpu_sc as plsc

VOCAB_MCC = 100000
VOCAB_TR = 1000
EMB = 128
B, T = 1024, 200
N = B * T

NC, NS = 2, 16
NW = NC * NS
PER_W = N // NW
C = 128
NCH = PER_W // C

_mesh = plsc.VectorSubcoreMesh(core_axis_name="c", subcore_axis_name="s")


@functools.partial(
    pl.kernel,
    out_type=jax.ShapeDtypeStruct((N, 2 * EMB), jnp.float32),
    mesh=_mesh,
    scratch_types=[
        pltpu.VMEM((C,), jnp.int32),
        pltpu.VMEM((C,), jnp.int32),
        pltpu.VMEM((C, EMB), jnp.float32),
        pltpu.VMEM((C, EMB), jnp.float32),
        pltpu.SemaphoreType.DMA,
        pltpu.SemaphoreType.DMA,
    ],
)
def _gather_concat(mcc_tab, tr_tab, idx_mcc, idx_tr, out,
                   idxm_v, idxt_v, rows_m, rows_t, sem_m, sem_t):
    wid = lax.axis_index("s") * NC + lax.axis_index("c")
    base = wid * PER_W

    def body(g, carry):
        off = pl.multiple_of(base + g * C, C)
        pltpu.sync_copy(idx_mcc.at[pl.ds(off, C)], idxm_v)
        pltpu.sync_copy(idx_tr.at[pl.ds(off, C)], idxt_v)
        for i in range(C // 16):
            s = pl.ds(i * 16, 16)
            vm = idxm_v[s]
            idxm_v[s] = jnp.minimum(jnp.maximum(vm, 0), VOCAB_MCC - 1)
            vt = idxt_v[s]
            idxt_v[s] = jnp.minimum(jnp.maximum(vt, 0), VOCAB_TR - 1)
        cm = pltpu.async_copy(mcc_tab.at[idxm_v], rows_m, sem_m)
        ct = pltpu.async_copy(tr_tab.at[idxt_v], rows_t, sem_t)
        cm.wait()
        ct.wait()
        pltpu.sync_copy(rows_m, out.at[pl.ds(off, C), pl.ds(0, EMB)])
        pltpu.sync_copy(rows_t, out.at[pl.ds(off, C), pl.ds(EMB, EMB)])
        return carry

    lax.fori_loop(0, NCH, body, 0)


def kernel(mcc_code, tr_type, emb_mcc, emb_tr):
    out = _gather_concat(emb_mcc, emb_tr,
                         mcc_code.reshape(N), tr_type.reshape(N))
    return out.reshape(B, T, 2 * EMB)


# speedup vs baseline: 7.5088x; 1.1794x over previous
import functools

import jax
import jax.numpy as jnp
from jax import lax
from jax.experimental import pallas as pl
from jax.experimental.pallas import tpu as pltpu
from jax.experimental.pallas import tpu_sc as plsc

VOCAB_MCC = 100000
VOCAB_TR = 1000
EMB = 128
B, T = 1024, 200
N = B * T

NC, NS = 2, 16
NW = NC * NS
PER_W = N // NW
C = 128
NCH = PER_W // C

_mesh = plsc.VectorSubcoreMesh(core_axis_name="c", subcore_axis_name="s")


@functools.partial(
    pl.kernel,
    out_type=jax.ShapeDtypeStruct((N, 2 * EMB), jnp.float32),
    mesh=_mesh,
    scratch_types=[
        pltpu.VMEM((PER_W,), jnp.int32),
        pltpu.VMEM((PER_W,), jnp.int32),
        pltpu.VMEM((C, EMB), jnp.float32),
        pltpu.VMEM((C, EMB), jnp.float32),
        pltpu.VMEM((C, EMB), jnp.float32),
        pltpu.VMEM((C, EMB), jnp.float32),
        pltpu.SemaphoreType.DMA,
        pltpu.SemaphoreType.DMA,
        pltpu.SemaphoreType.DMA,
        pltpu.SemaphoreType.DMA,
    ],
)
def _gather_concat(mcc_tab, tr_tab, idx_mcc, idx_tr, out,
                   idxm, idxt, rm0, rt0, rm1, rt1, gs0, gs1, ws0, ws1):
    wid = lax.axis_index("s") * NC + lax.axis_index("c")
    base = wid * PER_W
    pltpu.sync_copy(idx_mcc.at[pl.ds(base, PER_W)], idxm)
    pltpu.sync_copy(idx_tr.at[pl.ds(base, PER_W)], idxt)

    def clip(g):
        goff = g * C
        for i in range(C // 16):
            s = pl.ds(goff + i * 16, 16)
            idxm[s] = jnp.minimum(jnp.maximum(idxm[s], 0), VOCAB_MCC - 1)
            idxt[s] = jnp.minimum(jnp.maximum(idxt[s], 0), VOCAB_TR - 1)

    def g_desc(g, rm, rt, sem):
        sl = pl.ds(pl.multiple_of(g * C, C), C)
        return (pltpu.make_async_copy(mcc_tab.at[idxm.at[sl]], rm, sem),
                pltpu.make_async_copy(tr_tab.at[idxt.at[sl]], rt, sem))

    def w_desc(g, rm, rt, sem):
        off = pl.multiple_of(base + g * C, C)
        return (pltpu.make_async_copy(rm, out.at[pl.ds(off, C), pl.ds(0, EMB)], sem),
                pltpu.make_async_copy(rt, out.at[pl.ds(off, C), pl.ds(EMB, EMB)], sem))

    def fire(descs):
        for d in descs:
            d.start()

    def wait(descs):
        for d in descs:
            d.wait()

    clip(0)
    fire(g_desc(0, rm0, rt0, gs0))
    clip(1)
    fire(g_desc(1, rm1, rt1, gs1))

    def body(k, carry):
        g0 = 2 * k
        g1 = g0 + 1
        wait(g_desc(g0, rm0, rt0, gs0))
        fire(w_desc(g0, rm0, rt0, ws0))
        wait(g_desc(g1, rm1, rt1, gs1))
        fire(w_desc(g1, rm1, rt1, ws1))
        clip(g0 + 2)
        wait(w_desc(g0, rm0, rt0, ws0))
        fire(g_desc(g0 + 2, rm0, rt0, gs0))
        clip(g1 + 2)
        wait(w_desc(g1, rm1, rt1, ws1))
        fire(g_desc(g1 + 2, rm1, rt1, gs1))
        return carry

    lax.fori_loop(0, NCH // 2 - 1, body, 0)

    gl0, gl1 = NCH - 2, NCH - 1
    wait(g_desc(gl0, rm0, rt0, gs0))
    fire(w_desc(gl0, rm0, rt0, ws0))
    wait(g_desc(gl1, rm1, rt1, gs1))
    fire(w_desc(gl1, rm1, rt1, ws1))
    wait(w_desc(gl0, rm0, rt0, ws0))
    wait(w_desc(gl1, rm1, rt1, ws1))


def kernel(mcc_code, tr_type, emb_mcc, emb_tr):
    out = _gather_concat(emb_mcc, emb_tr,
                         mcc_code.reshape(N), tr_type.reshape(N))
    return out.reshape(B, T, 2 * EMB)


# speedup vs baseline: 8.1150x; 1.0807x over previous
import functools

import jax
import jax.numpy as jnp
from jax import lax
from jax.experimental import pallas as pl
from jax.experimental.pallas import tpu as pltpu
from jax.experimental.pallas import tpu_sc as plsc

VOCAB_MCC = 100000
VOCAB_TR = 1000
EMB = 128
B, T = 1024, 200
N = B * T

NC, NS = 2, 16
NW = NC * NS
PER_W = N // NW
C = 128
NCH = PER_W // C

_mesh = plsc.VectorSubcoreMesh(core_axis_name="c", subcore_axis_name="s")


@functools.partial(
    pl.kernel,
    out_type=jax.ShapeDtypeStruct((N, 2 * EMB), jnp.float32),
    mesh=_mesh,
    scratch_types=[
        pltpu.VMEM((PER_W,), jnp.int32),
        pltpu.VMEM((PER_W,), jnp.int32),
        [pltpu.VMEM((C, EMB), jnp.float32) for _ in range(3)],
        [pltpu.VMEM((C, EMB), jnp.float32) for _ in range(3)],
        [pltpu.SemaphoreType.DMA for _ in range(3)],
        [pltpu.SemaphoreType.DMA for _ in range(3)],
    ],
)
def _gather_concat(mcc_tab, tr_tab, idx_mcc, idx_tr, out,
                   idxm, idxt, rm, rt, gs, ws):
    wid = lax.axis_index("s") * NC + lax.axis_index("c")
    base = wid * PER_W
    pltpu.sync_copy(idx_mcc.at[pl.ds(base, PER_W)], idxm)
    pltpu.sync_copy(idx_tr.at[pl.ds(base, PER_W)], idxt)

    def clip(g):
        goff = g * C
        for i in range(C // 16):
            s = pl.ds(goff + i * 16, 16)
            idxm[s] = jnp.minimum(jnp.maximum(idxm[s], 0), VOCAB_MCC - 1)
            idxt[s] = jnp.minimum(jnp.maximum(idxt[s], 0), VOCAB_TR - 1)

    def g_desc(g, j, sem):
        sl = pl.ds(pl.multiple_of(g * C, C), C)
        return (pltpu.make_async_copy(mcc_tab.at[idxm.at[sl]], rm[j], sem),
                pltpu.make_async_copy(tr_tab.at[idxt.at[sl]], rt[j], sem))

    def w_desc(g, j, sem):
        off = pl.multiple_of(base + g * C, C)
        return (pltpu.make_async_copy(rm[j], out.at[pl.ds(off, C), pl.ds(0, EMB)], sem),
                pltpu.make_async_copy(rt[j], out.at[pl.ds(off, C), pl.ds(EMB, EMB)], sem))

    def fire(descs):
        for d in descs:
            d.start()

    def wait(descs):
        for d in descs:
            d.wait()

    def step(g, j, first=False, fire_next=True):
        wait(g_desc(g, j, gs[j]))
        fire(w_desc(g, j, ws[j]))
        jn = (j + 2) % 3
        if fire_next:
            clip(g + 2)
            if not first:
                wait(w_desc(g - 1, jn, ws[jn]))
            fire(g_desc(g + 2, jn, gs[jn]))
        elif not first:
            wait(w_desc(g - 1, jn, ws[jn]))

    clip(0)
    fire(g_desc(0, 0, gs[0]))
    clip(1)
    fire(g_desc(1, 1, gs[1]))
    step(0, 0, first=True)
    step(1, 1)
    step(2, 2)

    def body(k, carry):
        g = 3 * k + 3
        step(g, 0)
        step(g + 1, 1)
        step(g + 2, 2)
        return carry

    lax.fori_loop(0, (NCH - 6) // 3, body, 0)

    step(NCH - 2, (NCH - 2) % 3, fire_next=False)
    step(NCH - 1, (NCH - 1) % 3, fire_next=False)
    j = (NCH - 1) % 3
    wait(w_desc(NCH - 1, j, ws[j]))


def kernel(mcc_code, tr_type, emb_mcc, emb_tr):
    out = _gather_concat(emb_mcc, emb_tr,
                         mcc_code.reshape(N), tr_type.reshape(N))
    return out.reshape(B, T, 2 * EMB)
